# baseline (speedup 1.0000x reference)
import functools

import jax
import jax.numpy as jnp
from jax import lax
from jax.experimental import pallas as pl
import jax.experimental.pallas.tpu as pltpu
from jax.experimental.pallas import tpu_sc as plsc


def _tc_body(true_ref, pred_ref, omax_ref, otrue_ref, acc_max, acc_true, *,
             n_cols, col0, bc, nc):
    c = pl.program_id(1)

    @pl.when(c == 0)
    def _init():
        acc_max[...] = jnp.full_like(acc_max[...], -jnp.inf)
        acc_true[...] = jnp.zeros_like(acc_true[...])

    x = pred_ref[...]
    br = x.shape[0]
    cols = jax.lax.broadcasted_iota(jnp.int32, (br, bc), 1)
    t_local = true_ref[0] - (col0 + c * bc)
    hit = cols == t_local

    @pl.when(c < nc - 1)
    def _full_block():
        masked = jnp.where(hit, -jnp.inf, x)
        acc_max[...] = jnp.maximum(acc_max[...],
                                   jnp.max(masked, axis=1, keepdims=True))
        acc_true[...] = acc_true[...] + jnp.sum(
            jnp.where(hit, x, 0.0), axis=1, keepdims=True)

    @pl.when(c == nc - 1)
    def _edge_block():
        n_local = n_cols - col0 - c * bc
        masked = jnp.where(hit | (cols >= n_local), -jnp.inf, x)
        acc_max[...] = jnp.maximum(acc_max[...],
                                   jnp.max(masked, axis=1, keepdims=True))
        acc_true[...] = acc_true[...] + jnp.sum(
            jnp.where(hit & (cols < n_local), x, 0.0), axis=1, keepdims=True)

    @pl.when(c == nc - 1)
    def _finish():
        omax_ref[...] = acc_max[...]
        otrue_ref[...] = acc_true[...]


def _tc_call(n_rows, n_cols, col0, br, bc):
    nr = n_rows // br
    nc = -(-(n_cols - col0) // bc)
    body = functools.partial(_tc_body, n_cols=n_cols, col0=col0, bc=bc, nc=nc)
    return pl.pallas_call(
        body,
        grid=(nr, nc),
        in_specs=[
            pl.BlockSpec((1, br, 1), lambda r, c: (r, 0, 0)),
            pl.BlockSpec((br, bc), lambda r, c: (r, c + col0 // bc)),
        ],
        out_specs=[
            pl.BlockSpec((br, 1), lambda r, c: (r, 0)),
            pl.BlockSpec((br, 1), lambda r, c: (r, 0)),
        ],
        out_shape=[
            jax.ShapeDtypeStruct((n_rows, 1), jnp.float32),
            jax.ShapeDtypeStruct((n_rows, 1), jnp.float32),
        ],
        scratch_shapes=[
            pltpu.VMEM((br, 1), jnp.float32),
            pltpu.VMEM((br, 1), jnp.float32),
        ],
    )


_NW = 32
_LANES = 16
_GROUP = 8
_CH = 5120
_UNROLL = 2


def _sc_partial_call(n_rows, sc_cols, rpw):
    mesh = plsc.VectorSubcoreMesh(core_axis_name="c", subcore_axis_name="s")
    n_chunks = sc_cols // _CH

    @functools.partial(
        pl.kernel, mesh=mesh,
        out_type=jax.ShapeDtypeStruct((n_rows * _LANES,), jnp.float32),
        scratch_types=[
            pltpu.VMEM((_GROUP, _CH), jnp.float32),
            pltpu.VMEM((_GROUP, _CH), jnp.float32),
            pltpu.VMEM((rpw * _LANES,), jnp.int32),
            pltpu.VMEM((_LANES,), jnp.float32),
            pltpu.VMEM((2 * _LANES,), jnp.float32),
            pltpu.SemaphoreType.DMA,
            pltpu.SemaphoreType.DMA,
        ])
    def k(pred_hbm, taux_hbm, out_hbm, buf0, buf1, tb, ovec, sbuf,
          sem0, sem1):
        wid = lax.axis_index("s") * 2 + lax.axis_index("c")
        base = wid * rpw
        pltpu.sync_copy(taux_hbm.at[pl.ds(base * _LANES, rpw * _LANES)], tb)

        lane = lax.iota(jnp.int32, _LANES)
        lane15 = lane == (_LANES - 1)
        lane14 = lane == (_LANES - 2)
        neg = jnp.full((_LANES,), -jnp.inf, jnp.float32)
        zero = jnp.zeros((_LANES,), jnp.float32)
        bufs = (buf0, buf1)
        sems = (sem0, sem1)

        def _rotate_reduce(vec, op):
            for s in (8, 4, 2, 1):
                sbuf[pl.ds(0, _LANES)] = vec
                sbuf[pl.ds(_LANES, _LANES)] = vec
                vec = op(vec, sbuf[pl.ds(s, _LANES)])
            return vec

        def group_body(grp, carry):
            rowb = pl.multiple_of(base + grp * _GROUP, _GROUP)
            rel0 = grp * _GROUP
            tvs = [tb[pl.ds((rel0 + r) * _LANES, _LANES)]
                   for r in range(_GROUP)]

            def fire(c):
                return pltpu.async_copy(
                    pred_hbm.at[pl.ds(rowb, _GROUP), pl.ds(c * _CH, _CH)],
                    bufs[c % 2], sems[c % 2])

            copies = [fire(0)]
            maccs = [neg] * _GROUP
            taccs = [zero] * _GROUP
            for c in range(n_chunks):
                if c + 1 < n_chunks:
                    copies.append(fire(c + 1))
                copies[c].wait()
                buf = bufs[c % 2]
                t_locs = [tv - c * _CH for tv in tvs]

                def chunk_body(i, cr, _buf=buf, _tl=t_locs):
                    ms = list(cr[0])
                    ts = list(cr[1])
                    cb = i * (_UNROLL * _LANES)
                    for r in range(_GROUP):
                        for u in range(_UNROLL):
                            v = _buf[r, pl.ds(cb + u * _LANES, _LANES)]
                            col = lane + (cb + u * _LANES)
                            hit = col == _tl[r]
                            ms[r] = jnp.maximum(
                                ms[r], jnp.where(hit, neg, v))
                            ts[r] = ts[r] + jnp.where(hit, v, zero)
                    return (tuple(ms), tuple(ts))

                maccs, taccs = lax.fori_loop(
                    0, _CH // (_UNROLL * _LANES), chunk_body,
                    (tuple(maccs), tuple(taccs)))
                maccs = list(maccs)
                taccs = list(taccs)

            for r in range(_GROUP):
                m_all = _rotate_reduce(maccs[r], jnp.maximum)
                t_all = _rotate_reduce(taccs[r], jnp.add)
                packed = (jnp.where(lane15, m_all, zero)
                          + jnp.where(lane14, t_all, zero))
                ovec[...] = packed
                pltpu.sync_copy(
                    ovec, out_hbm.at[pl.ds((rowb + r) * _LANES, _LANES)])
            return carry

        lax.fori_loop(0, rpw // _GROUP, group_body, 0)

    return k


def _combine_body(tcm_ref, tct_ref, scp_ref, out_ref):
    m = jnp.maximum(tcm_ref[...], scp_ref[:, 15:16])
    t = tct_ref[...] + scp_ref[:, 14:15]
    out_ref[...] = jnp.sum(m - t, keepdims=True)


def _combine_call(n_rows):
    return pl.pallas_call(
        _combine_body,
        out_shape=jax.ShapeDtypeStruct((1, 1), jnp.float32),
    )


_BR = 1024
_BC = 2048
_SC_COLS = 61440


@jax.jit
def kernel(pred, true):
    n_rows, n_cols = pred.shape
    rpw = n_rows // _NW
    true3d = true.reshape(n_rows // _BR, _BR, 1)
    taux = jnp.broadcast_to(true[:, None], (n_rows, _LANES)).reshape(-1)
    sc_flat = _sc_partial_call(n_rows, _SC_COLS, rpw)(pred, taux)
    tc_max, tc_true = _tc_call(n_rows, n_cols, _SC_COLS, _BR, _BC)(
        true3d, pred)
    scp = sc_flat.reshape(n_rows, _LANES)
    out = _combine_call(n_rows)(tc_max, tc_true, scp)
    return out[0, 0]


# speedup vs baseline: 1.0629x; 1.0629x over previous
import functools

import jax
import jax.numpy as jnp
from jax.experimental import pallas as pl
import jax.experimental.pallas.tpu as pltpu


def _mismatch_body(true_ref, pred_ref, out_ref, acc_max, acc_true, *, n_cols,
                   bc, nc):
    r = pl.program_id(0)
    c = pl.program_id(1)

    @pl.when(c == 0)
    def _init():
        acc_max[...] = jnp.full_like(acc_max[...], -jnp.inf)
        acc_true[...] = jnp.zeros_like(acc_true[...])

    x = pred_ref[...]
    br = x.shape[0]
    cols = jax.lax.broadcasted_iota(jnp.int32, (br, bc), 1)
    t_local = true_ref[0] - c * bc
    hit = cols == t_local

    @pl.when(c < nc - 1)
    def _full_block():
        masked = jnp.where(hit, -jnp.inf, x)
        acc_max[...] = jnp.maximum(acc_max[...],
                                   jnp.max(masked, axis=1, keepdims=True))
        acc_true[...] = acc_true[...] + jnp.sum(
            jnp.where(hit, x, 0.0), axis=1, keepdims=True)

    @pl.when(c == nc - 1)
    def _edge_block():
        n_local = n_cols - c * bc
        masked = jnp.where(hit | (cols >= n_local), -jnp.inf, x)
        acc_max[...] = jnp.maximum(acc_max[...],
                                   jnp.max(masked, axis=1, keepdims=True))
        acc_true[...] = acc_true[...] + jnp.sum(
            jnp.where(hit & (cols < n_local), x, 0.0), axis=1, keepdims=True)

    @pl.when(c == nc - 1)
    def _finish():
        part = jnp.sum(acc_max[...] - acc_true[...], keepdims=True)

        @pl.when(r == 0)
        def _first():
            out_ref[...] = part

        @pl.when(r != 0)
        def _rest():
            out_ref[...] = out_ref[...] + part


def _build_call(n_rows, n_cols, br, bc, interpret=False):
    nr = n_rows // br
    nc = -(-n_cols // bc)
    body = functools.partial(_mismatch_body, n_cols=n_cols, bc=bc, nc=nc)
    return pl.pallas_call(
        body,
        grid=(nr, nc),
        in_specs=[
            pl.BlockSpec((1, br, 1), lambda r, c: (r, 0, 0)),
            pl.BlockSpec((br, bc), lambda r, c: (r, c)),
        ],
        out_specs=pl.BlockSpec((1, 1), lambda r, c: (0, 0)),
        out_shape=jax.ShapeDtypeStruct((1, 1), jnp.float32),
        scratch_shapes=[
            pltpu.VMEM((br, 1), jnp.float32),
            pltpu.VMEM((br, 1), jnp.float32),
        ],
        interpret=interpret,
    )


_BR = 2048
_BC = 2048


@jax.jit
def kernel(pred, true):
    n_rows, n_cols = pred.shape
    br = _BR
    bc = _BC
    call = _build_call(n_rows, n_cols, br, bc)
    out = call(true.reshape(n_rows // br, br, 1), pred)
    return out[0, 0]
